# baseline (speedup 1.0000x reference)
import jax
import jax.numpy as jnp
from jax import lax
from jax.experimental import pallas as pl
from jax.experimental.pallas import tpu as pltpu
from jax.experimental.pallas import tpu_sc as plsc

N = 10000
E = 320000
F = 128
H = 64
EROWS = E // 128
BN_EPS = 1e-5

_ZCH = 640
_BASE_ROWS = EROWS // 16
_EXTRA = EROWS - 16 * _BASE_ROWS

import functools


@functools.cache
def _sc_mesh():
    return plsc.VectorSubcoreMesh(core_axis_name="c", subcore_axis_name="s",
                                  num_cores=2, num_subcores=16)


NP = N + 16
EPROWS = 2560
RPT = EPROWS // 16
NBUF = 3
CROWS = 1
NCH = RPT // CROWS


def _mp_pass(xsrc, xdst, srcbuf, dstbuf, rbufs, gsems, ssems):
    del ssems

    def gidx(buf, k):
        if CROWS == 1:
            return buf.at[k]
        return buf.at[pl.ds(CROWS * k, CROWS)]

    for b in range(NBUF):
        pltpu.async_copy(xsrc.at[gidx(srcbuf, b)], rbufs[b], gsems[b])

    def outer(q, carry):
        for b in range(NBUF):
            k = q * NBUF + b
            pltpu.make_async_copy(xsrc.at[gidx(srcbuf, k)], rbufs[b],
                                  gsems[b]).wait()

            @pl.when(k < NCH - NBUF)
            def _():
                pltpu.async_copy(xsrc.at[gidx(srcbuf, k + NBUF)], rbufs[b],
                                 gsems[b])
        return carry

    lax.fori_loop(0, NCH // NBUF, outer, 0)
    for k in range((NCH // NBUF) * NBUF, NCH):
        b = k % NBUF
        pltpu.make_async_copy(xsrc.at[gidx(srcbuf, k)], rbufs[b],
                              gsems[b]).wait()
    plsc.subcore_barrier()


def _seg_chain_body(srcp, dstp, x_hbm, xwide, zeros_hbm, y1, y2, y3, y4,
                    srcbuf, dstbuf, r0, r1, r2, accum,
                    g0, g1, g2):
    c = lax.axis_index("c")
    s = lax.axis_index("s")
    cN = c * N
    rbufs = [r0, r1, r2]
    gsems = [g0, g1, g2]
    ssems = None

    pltpu.sync_copy(srcp.at[pl.ds(s * RPT, RPT)], srcbuf)
    pltpu.sync_copy(dstp.at[pl.ds(s * RPT, RPT)], dstbuf)

    def shift(r, carry):
        for i in range(128 // 16):
            srcbuf[r, pl.ds(i * 16, 16)] = srcbuf[r, pl.ds(i * 16, 16)] + cN
        return carry

    lax.fori_loop(0, RPT, shift, 0)

    @pl.when(s < 15)
    def _():
        pltpu.sync_copy(zeros_hbm.at[pl.ds(s * 640, 640)],
                        accum.at[pl.ds(s * 640, 640)])

    @pl.when(s == 15)
    def _():
        pltpu.sync_copy(zeros_hbm.at[pl.ds(9600, 416)],
                        accum.at[pl.ds(9600, 416)])

    plsc.subcore_barrier()

    def flush(yout, last=False):
        @pl.when(s < 15)
        def _():
            pltpu.sync_copy(accum.at[pl.ds(s * 640, 640)],
                            yout.at[pl.ds(cN + s * 640, 640)])
            if not last:
                pltpu.sync_copy(zeros_hbm.at[pl.ds(s * 640, 640)],
                                accum.at[pl.ds(s * 640, 640)])

        @pl.when(s == 15)
        def _():
            pltpu.sync_copy(accum.at[pl.ds(9600, 400)],
                            yout.at[pl.ds(cN + 9600, 400)])
            if not last:
                pltpu.sync_copy(zeros_hbm.at[pl.ds(9600, 416)],
                                accum.at[pl.ds(9600, 416)])

        plsc.subcore_barrier()

    _mp_pass(xwide, accum, srcbuf, dstbuf, rbufs, gsems, ssems)
    flush(y1)
    _mp_pass(xwide, accum, srcbuf, dstbuf, rbufs, gsems, ssems)
    flush(y2)
    _mp_pass(xwide, accum, srcbuf, dstbuf, rbufs, gsems, ssems)
    flush(y3)
    _mp_pass(xwide, accum, srcbuf, dstbuf, rbufs, gsems, ssems)
    flush(y4, last=True)


@functools.cache
def _seg_chain_kernel():
    ytype = jax.ShapeDtypeStruct((2 * N, H), jnp.float32)
    return pl.kernel(
        _seg_chain_body,
        out_type=[ytype, ytype, ytype, ytype],
        mesh=_sc_mesh(),
        scratch_types=[
            pltpu.VMEM((RPT, 128), jnp.int32),
            pltpu.VMEM((RPT, 128), jnp.int32),
        ] + [pltpu.VMEM((CROWS * 128, 128), jnp.float32)] * 3 + [
            pltpu.VMEM_SHARED((NP, H), jnp.float32),
        ] + [pltpu.SemaphoreType.DMA] * 3,
        compiler_params=pltpu.CompilerParams(use_tc_tiling_on_sc=False),
        cost_estimate=pl.CostEstimate(flops=170_000_000,
                                      bytes_accessed=1_300_000_000,
                                      transcendentals=0),
    )


def _seg_chain(srcp, dstp, x_hbm, xwide, zeros_hbm):
    y1, y2, _, y4 = _seg_chain_kernel()(srcp, dstp, x_hbm, xwide, zeros_hbm)
    return y1, y2, y4


_RB = 1000
_AB = 200


def _g_body(fb_ref, wt_ref, o_ref):
    o_ref[...] = jnp.dot(fb_ref[...], wt_ref[...],
                         preferred_element_type=jnp.float32)


def _compute_g(feat_b, wt):
    return pl.pallas_call(
        _g_body,
        grid=(N // _RB,),
        in_specs=[pl.BlockSpec((_RB, F), lambda i: (i, 0)),
                  pl.BlockSpec((F, F), lambda i: (0, 0))],
        out_specs=pl.BlockSpec((_RB, F), lambda i: (i, 0)),
        out_shape=jax.ShapeDtypeStruct((N, F), jnp.float32),
    )(feat_b, wt)


def _dense_body(pm, g, fa, dg, wpt, wdt, bsum, o):
    acc = jnp.dot(pm[...].astype(jnp.bfloat16), g[...].astype(jnp.bfloat16),
                  preferred_element_type=jnp.float32)
    o[...] = (acc
              + jnp.dot(fa[...], wpt[...], preferred_element_type=jnp.float32)
              + jnp.dot(dg[...] * fa[...], wdt[...],
                        preferred_element_type=jnp.float32)
              + bsum[...])


def _compute_dense(pm_pd, g, feat_a, deg, wpt, wdt, bsum):
    return pl.pallas_call(
        _dense_body,
        grid=(N // _AB,),
        in_specs=[
            pl.BlockSpec((_AB, N), lambda i: (i, 0)),
            pl.BlockSpec((N, F), lambda i: (0, 0)),
            pl.BlockSpec((_AB, F), lambda i: (i, 0)),
            pl.BlockSpec((_AB, 1), lambda i: (i, 0)),
            pl.BlockSpec((F, F), lambda i: (0, 0)),
            pl.BlockSpec((F, F), lambda i: (0, 0)),
            pl.BlockSpec((1, F), lambda i: (0, 0)),
        ],
        out_specs=pl.BlockSpec((_AB, F), lambda i: (i, 0)),
        out_shape=jax.ShapeDtypeStruct((N, F), jnp.float32),
    )(pm_pd, g, feat_a, deg, wpt, wdt, bsum)


def _radius_body(dn, y1l, y1h, y2l, y2h, y4l, y4h,
                 w0l, w0h, w1l, w1h, w2l, w2h, raw, sums, sumsq):
    r = dn[...]
    r += jnp.dot(y1l[...], w0l[...], preferred_element_type=jnp.float32)
    r += jnp.dot(y1h[...], w0h[...], preferred_element_type=jnp.float32)
    r += jnp.dot(y2l[...], w1l[...], preferred_element_type=jnp.float32)
    r += jnp.dot(y2h[...], w1h[...], preferred_element_type=jnp.float32)
    r += jnp.dot(y4l[...], w2l[...], preferred_element_type=jnp.float32)
    r += jnp.dot(y4h[...], w2h[...], preferred_element_type=jnp.float32)
    raw[...] = r
    sums[0] = jnp.sum(r, axis=0, keepdims=True)
    sumsq[0] = jnp.sum(r * r, axis=0, keepdims=True)


def _compute_radius(dense, y1, y2, y4, wmats):
    nb = N // _RB
    yspec_l = pl.BlockSpec((_RB, H), lambda i: (i, 0))
    yspec_h = pl.BlockSpec((_RB, H), lambda i: (i + nb, 0))
    wspec = pl.BlockSpec((H, F), lambda i: (0, 0))
    return pl.pallas_call(
        _radius_body,
        grid=(nb,),
        in_specs=[pl.BlockSpec((_RB, F), lambda i: (i, 0)),
                  yspec_l, yspec_h, yspec_l, yspec_h, yspec_l, yspec_h,
                  wspec, wspec, wspec, wspec, wspec, wspec],
        out_specs=[pl.BlockSpec((_RB, F), lambda i: (i, 0)),
                   pl.BlockSpec((1, 1, F), lambda i: (i, 0, 0)),
                   pl.BlockSpec((1, 1, F), lambda i: (i, 0, 0))],
        out_shape=[jax.ShapeDtypeStruct((N, F), jnp.float32),
                   jax.ShapeDtypeStruct((nb, 1, F), jnp.float32),
                   jax.ShapeDtypeStruct((nb, 1, F), jnp.float32)],
    )(dense, y1, y1, y2, y2, y4, y4, *wmats)


def _bn_body(raw, sums, sumsq, gm, bt, o):
    S = jnp.sum(sums[...], axis=0)
    Q = jnp.sum(sumsq[...], axis=0)
    mean = S / N
    var = Q / N - mean * mean
    rstd = lax.rsqrt(var + BN_EPS)
    scale = gm[...] * rstd
    o[...] = raw[...] * scale + (bt[...] - mean * scale)


def _compute_bn(raw, sums, sumsq, gamma, beta):
    nb = N // _RB
    return pl.pallas_call(
        _bn_body,
        grid=(nb,),
        in_specs=[pl.BlockSpec((_RB, F), lambda i: (i, 0)),
                  pl.BlockSpec((nb, 1, F), lambda i: (0, 0, 0)),
                  pl.BlockSpec((nb, 1, F), lambda i: (0, 0, 0)),
                  pl.BlockSpec((1, F), lambda i: (0, 0)),
                  pl.BlockSpec((1, F), lambda i: (0, 0))],
        out_specs=pl.BlockSpec((_RB, F), lambda i: (i, 0)),
        out_shape=jax.ShapeDtypeStruct((N, F), jnp.float32),
    )(raw, sums, sumsq, gamma, beta)


def kernel(edge_index, feat_a, feat_b, deg, pm_pd, W_prev, b_prev, W_deg, b_deg,
           W_rad_0, b_rad_0, W_rad_1, b_rad_1, W_rad_2, b_rad_2,
           W_fuse, b_fuse, bn_gamma, bn_beta):
    npad = EPROWS * 128 - E
    pad_src = (jnp.arange(npad, dtype=jnp.int32) * 37) % N
    pad_dst = N + (jnp.arange(npad, dtype=jnp.int32) % 16)
    srcp = jnp.concatenate([edge_index[0].astype(jnp.int32), pad_src]
                           ).reshape(EPROWS, 128)
    dstp = jnp.concatenate([edge_index[1].astype(jnp.int32), pad_dst]
                           ).reshape(EPROWS, 128)

    x0 = jnp.concatenate([feat_a[:, :H], feat_a[:, H:]], axis=0)
    zeros = jnp.zeros((NP, H), jnp.float32)

    xwide = jnp.concatenate([feat_a, feat_a], axis=0)
    y1, y2, y4 = _seg_chain(srcp, dstp, x0, xwide, zeros)

    g = _compute_g(feat_b, W_fuse.T)
    bsum = (b_prev + b_deg + b_fuse + b_rad_0 + b_rad_1 + b_rad_2)[None, :]
    dense = _compute_dense(pm_pd, g, feat_a, deg, W_prev.T, W_deg.T, bsum)

    wmats = [W_rad_0[:, :H].T, W_rad_0[:, H:].T,
             W_rad_1[:, :H].T, W_rad_1[:, H:].T,
             W_rad_2[:, :H].T, W_rad_2[:, H:].T]
    raw, sums, sumsq = _compute_radius(dense, y1, y2, y4, wmats)

    return _compute_bn(raw, sums, sumsq, bn_gamma[None, :], bn_beta[None, :])


# speedup vs baseline: 1.4969x; 1.4969x over previous
import jax
import jax.numpy as jnp
from jax import lax
from jax.experimental import pallas as pl
from jax.experimental.pallas import tpu as pltpu
from jax.experimental.pallas import tpu_sc as plsc

N = 10000
E = 320000
F = 128
H = 64
EROWS = E // 128
BN_EPS = 1e-5

_ZCH = 640
_BASE_ROWS = EROWS // 16
_EXTRA = EROWS - 16 * _BASE_ROWS

import functools


@functools.cache
def _sc_mesh():
    return plsc.VectorSubcoreMesh(core_axis_name="c", subcore_axis_name="s",
                                  num_cores=2, num_subcores=16)


NP = N + 16
EPROWS = 2560
RPT = EPROWS // 16
NBUF = 4
CROWS = 1
NCH = RPT // CROWS


def _mp_pass(xsrc, xdst, srcbuf, dstbuf, rbufs, gsems, ssems):
    del ssems

    def gidx(buf, k):
        if CROWS == 1:
            return buf.at[k]
        return buf.at[pl.ds(CROWS * k, CROWS)]

    for b in range(NBUF):
        pltpu.async_copy(xsrc.at[gidx(srcbuf, b)], rbufs[b], gsems[b])

    def outer(q, carry):
        for b in range(NBUF):
            k = q * NBUF + b
            pltpu.make_async_copy(xsrc.at[gidx(srcbuf, k)], rbufs[b],
                                  gsems[b]).wait()
            pltpu.sync_copy(rbufs[b], xdst.at[gidx(dstbuf, k)], add=True)

            @pl.when(k < NCH - NBUF)
            def _():
                pltpu.async_copy(xsrc.at[gidx(srcbuf, k + NBUF)], rbufs[b],
                                 gsems[b])
        return carry

    lax.fori_loop(0, NCH // NBUF, outer, 0)
    for k in range((NCH // NBUF) * NBUF, NCH):
        b = k % NBUF
        pltpu.make_async_copy(xsrc.at[gidx(srcbuf, k)], rbufs[b],
                              gsems[b]).wait()
        pltpu.sync_copy(rbufs[b], xdst.at[gidx(dstbuf, k)], add=True)
    plsc.subcore_barrier()


def _seg_chain_body(srcp, dstp, x_hbm, zeros_hbm, y1, y2, y3, y4,
                    srcbuf, dstbuf, r0, r1, r2, r3, accum,
                    g0, g1, g2, g3):
    c = lax.axis_index("c")
    s = lax.axis_index("s")
    cN = c * N
    rbufs = [r0, r1, r2, r3]
    gsems = [g0, g1, g2, g3]
    ssems = None

    pltpu.sync_copy(srcp.at[pl.ds(s * RPT, RPT)], srcbuf)
    pltpu.sync_copy(dstp.at[pl.ds(s * RPT, RPT)], dstbuf)

    def shift(r, carry):
        for i in range(128 // 16):
            srcbuf[r, pl.ds(i * 16, 16)] = srcbuf[r, pl.ds(i * 16, 16)] + cN
        return carry

    lax.fori_loop(0, RPT, shift, 0)

    @pl.when(s < 15)
    def _():
        pltpu.sync_copy(zeros_hbm.at[pl.ds(s * 640, 640)],
                        accum.at[pl.ds(s * 640, 640)])

    @pl.when(s == 15)
    def _():
        pltpu.sync_copy(zeros_hbm.at[pl.ds(9600, 416)],
                        accum.at[pl.ds(9600, 416)])

    plsc.subcore_barrier()

    def flush(yout, last=False):
        @pl.when(s < 15)
        def _():
            pltpu.sync_copy(accum.at[pl.ds(s * 640, 640)],
                            yout.at[pl.ds(cN + s * 640, 640)])
            if not last:
                pltpu.sync_copy(zeros_hbm.at[pl.ds(s * 640, 640)],
                                accum.at[pl.ds(s * 640, 640)])

        @pl.when(s == 15)
        def _():
            pltpu.sync_copy(accum.at[pl.ds(9600, 400)],
                            yout.at[pl.ds(cN + 9600, 400)])
            if not last:
                pltpu.sync_copy(zeros_hbm.at[pl.ds(9600, 416)],
                                accum.at[pl.ds(9600, 416)])

        plsc.subcore_barrier()

    _mp_pass(x_hbm, accum, srcbuf, dstbuf, rbufs, gsems, ssems)
    flush(y1)
    _mp_pass(y1, accum, srcbuf, dstbuf, rbufs, gsems, ssems)
    flush(y2)
    _mp_pass(y2, accum, srcbuf, dstbuf, rbufs, gsems, ssems)
    flush(y3)
    _mp_pass(y3, accum, srcbuf, dstbuf, rbufs, gsems, ssems)
    flush(y4, last=True)


@functools.cache
def _seg_chain_kernel():
    ytype = jax.ShapeDtypeStruct((2 * N, H), jnp.float32)
    return pl.kernel(
        _seg_chain_body,
        out_type=[ytype, ytype, ytype, ytype],
        mesh=_sc_mesh(),
        scratch_types=[
            pltpu.VMEM((RPT, 128), jnp.int32),
            pltpu.VMEM((RPT, 128), jnp.int32),
        ] + [pltpu.VMEM((CROWS * 128, H), jnp.float32)] * 4 + [
            pltpu.VMEM_SHARED((NP, H), jnp.float32),
        ] + [pltpu.SemaphoreType.DMA] * 4,
        compiler_params=pltpu.CompilerParams(use_tc_tiling_on_sc=False),
        cost_estimate=pl.CostEstimate(flops=170_000_000,
                                      bytes_accessed=1_300_000_000,
                                      transcendentals=0),
    )


def _seg_chain(srcp, dstp, x_hbm, zeros_hbm):
    y1, y2, _, y4 = _seg_chain_kernel()(srcp, dstp, x_hbm, zeros_hbm)
    return y1, y2, y4


_RB = 1000
_AB = 200


def _g_body(fb_ref, wt_ref, o_ref):
    o_ref[...] = jnp.dot(fb_ref[...], wt_ref[...],
                         preferred_element_type=jnp.float32)


def _compute_g(feat_b, wt):
    return pl.pallas_call(
        _g_body,
        grid=(N // _RB,),
        in_specs=[pl.BlockSpec((_RB, F), lambda i: (i, 0)),
                  pl.BlockSpec((F, F), lambda i: (0, 0))],
        out_specs=pl.BlockSpec((_RB, F), lambda i: (i, 0)),
        out_shape=jax.ShapeDtypeStruct((N, F), jnp.float32),
    )(feat_b, wt)


def _dense_body(pm, g, fa, dg, wpt, wdt, bsum, o):
    acc = jnp.dot(pm[...].astype(jnp.bfloat16), g[...].astype(jnp.bfloat16),
                  preferred_element_type=jnp.float32)
    o[...] = (acc
              + jnp.dot(fa[...], wpt[...], preferred_element_type=jnp.float32)
              + jnp.dot(dg[...] * fa[...], wdt[...],
                        preferred_element_type=jnp.float32)
              + bsum[...])


def _compute_dense(pm_pd, g, feat_a, deg, wpt, wdt, bsum):
    return pl.pallas_call(
        _dense_body,
        grid=(N // _AB,),
        in_specs=[
            pl.BlockSpec((_AB, N), lambda i: (i, 0)),
            pl.BlockSpec((N, F), lambda i: (0, 0)),
            pl.BlockSpec((_AB, F), lambda i: (i, 0)),
            pl.BlockSpec((_AB, 1), lambda i: (i, 0)),
            pl.BlockSpec((F, F), lambda i: (0, 0)),
            pl.BlockSpec((F, F), lambda i: (0, 0)),
            pl.BlockSpec((1, F), lambda i: (0, 0)),
        ],
        out_specs=pl.BlockSpec((_AB, F), lambda i: (i, 0)),
        out_shape=jax.ShapeDtypeStruct((N, F), jnp.float32),
    )(pm_pd, g, feat_a, deg, wpt, wdt, bsum)


def _radius_body(dn, y1l, y1h, y2l, y2h, y4l, y4h,
                 w0l, w0h, w1l, w1h, w2l, w2h, raw, sums, sumsq):
    r = dn[...]
    r += jnp.dot(y1l[...], w0l[...], preferred_element_type=jnp.float32)
    r += jnp.dot(y1h[...], w0h[...], preferred_element_type=jnp.float32)
    r += jnp.dot(y2l[...], w1l[...], preferred_element_type=jnp.float32)
    r += jnp.dot(y2h[...], w1h[...], preferred_element_type=jnp.float32)
    r += jnp.dot(y4l[...], w2l[...], preferred_element_type=jnp.float32)
    r += jnp.dot(y4h[...], w2h[...], preferred_element_type=jnp.float32)
    raw[...] = r
    sums[0] = jnp.sum(r, axis=0, keepdims=True)
    sumsq[0] = jnp.sum(r * r, axis=0, keepdims=True)


def _compute_radius(dense, y1, y2, y4, wmats):
    nb = N // _RB
    yspec_l = pl.BlockSpec((_RB, H), lambda i: (i, 0))
    yspec_h = pl.BlockSpec((_RB, H), lambda i: (i + nb, 0))
    wspec = pl.BlockSpec((H, F), lambda i: (0, 0))
    return pl.pallas_call(
        _radius_body,
        grid=(nb,),
        in_specs=[pl.BlockSpec((_RB, F), lambda i: (i, 0)),
                  yspec_l, yspec_h, yspec_l, yspec_h, yspec_l, yspec_h,
                  wspec, wspec, wspec, wspec, wspec, wspec],
        out_specs=[pl.BlockSpec((_RB, F), lambda i: (i, 0)),
                   pl.BlockSpec((1, 1, F), lambda i: (i, 0, 0)),
                   pl.BlockSpec((1, 1, F), lambda i: (i, 0, 0))],
        out_shape=[jax.ShapeDtypeStruct((N, F), jnp.float32),
                   jax.ShapeDtypeStruct((nb, 1, F), jnp.float32),
                   jax.ShapeDtypeStruct((nb, 1, F), jnp.float32)],
    )(dense, y1, y1, y2, y2, y4, y4, *wmats)


def _bn_body(raw, sums, sumsq, gm, bt, o):
    S = jnp.sum(sums[...], axis=0)
    Q = jnp.sum(sumsq[...], axis=0)
    mean = S / N
    var = Q / N - mean * mean
    rstd = lax.rsqrt(var + BN_EPS)
    scale = gm[...] * rstd
    o[...] = raw[...] * scale + (bt[...] - mean * scale)


def _compute_bn(raw, sums, sumsq, gamma, beta):
    nb = N // _RB
    return pl.pallas_call(
        _bn_body,
        grid=(nb,),
        in_specs=[pl.BlockSpec((_RB, F), lambda i: (i, 0)),
                  pl.BlockSpec((nb, 1, F), lambda i: (0, 0, 0)),
                  pl.BlockSpec((nb, 1, F), lambda i: (0, 0, 0)),
                  pl.BlockSpec((1, F), lambda i: (0, 0)),
                  pl.BlockSpec((1, F), lambda i: (0, 0))],
        out_specs=pl.BlockSpec((_RB, F), lambda i: (i, 0)),
        out_shape=jax.ShapeDtypeStruct((N, F), jnp.float32),
    )(raw, sums, sumsq, gamma, beta)


def kernel(edge_index, feat_a, feat_b, deg, pm_pd, W_prev, b_prev, W_deg, b_deg,
           W_rad_0, b_rad_0, W_rad_1, b_rad_1, W_rad_2, b_rad_2,
           W_fuse, b_fuse, bn_gamma, bn_beta):
    npad = EPROWS * 128 - E
    pad_src = (jnp.arange(npad, dtype=jnp.int32) * 37) % N
    pad_dst = N + (jnp.arange(npad, dtype=jnp.int32) % 16)
    srcp = jnp.concatenate([edge_index[0].astype(jnp.int32), pad_src]
                           ).reshape(EPROWS, 128)
    dstp = jnp.concatenate([edge_index[1].astype(jnp.int32), pad_dst]
                           ).reshape(EPROWS, 128)

    x0 = jnp.concatenate([feat_a[:, :H], feat_a[:, H:]], axis=0)
    zeros = jnp.zeros((NP, H), jnp.float32)

    y1, y2, y4 = _seg_chain(srcp, dstp, x0, zeros)

    g = _compute_g(feat_b, W_fuse.T)
    bsum = (b_prev + b_deg + b_fuse + b_rad_0 + b_rad_1 + b_rad_2)[None, :]
    dense = _compute_dense(pm_pd, g, feat_a, deg, W_prev.T, W_deg.T, bsum)

    wmats = [W_rad_0[:, :H].T, W_rad_0[:, H:].T,
             W_rad_1[:, :H].T, W_rad_1[:, H:].T,
             W_rad_2[:, :H].T, W_rad_2[:, H:].T]
    raw, sums, sumsq = _compute_radius(dense, y1, y2, y4, wmats)

    return _compute_bn(raw, sums, sumsq, bn_gamma[None, :], bn_beta[None, :])
